# speedup vs baseline: 5.1351x; 5.1351x over previous
import jax
import jax.numpy as jnp
from jax.experimental import pallas as pl
from jax.experimental.pallas import tpu as pltpu

_GAMMA = 2.0
_ALPHA = 0.25
_THR_POS = 0.05
_THR_NEG = 0.02

_B, _H, _W, _NB = 16, 128, 128, 1024
_N_PIX = _B * _H * _W


def _roll(x, shift):
    return jnp.roll(x, shift, axis=1)


def _loss_kernel(cl_ref, re_ref, gr_ref, gt_ref, cf_ref, op_ref, bb_ref,
                 br_ref, gbt_ref, out_ref):
    b = pl.program_id(0)
    f32 = jnp.float32

    @pl.when(b == 0)
    def _init():
        for i in range(8):
            out_ref[0, i] = 0.0

    T = gt_ref[0].astype(f32)

    X = cl_ref[0]
    Xs = _roll(X, -1)
    lse = jnp.maximum(X, Xs) + jnp.log1p(jnp.exp(-jnp.abs(X - Xs)))
    col2 = jax.lax.broadcasted_iota(jnp.int32, (_H, 2 * _W), 1)
    row2 = jax.lax.broadcasted_iota(jnp.int32, (_H, 2 * _W), 0)
    E2 = (col2 == 2 * row2).astype(jnp.bfloat16)
    T2 = jnp.dot(T.astype(jnp.bfloat16), E2, preferred_element_type=f32)
    xt = jnp.where(T2 >= 0.5, X, Xs)
    logpt = xt - lse
    pt = jnp.exp(logpt)
    at = jnp.where(T2 >= 0.5, _ALPHA, 1.0 - _ALPHA)
    om = 1.0 - pt
    term = -at * om * om * logpt
    rpn0_s = jnp.sum(jnp.where(col2 % 2 == 0, term, 0.0))

    R = re_ref[0]
    G = gr_ref[0]
    col4 = jax.lax.broadcasted_iota(jnp.int32, (_H, 4 * _W), 1)
    row4 = jax.lax.broadcasted_iota(jnp.int32, (_H, 4 * _W), 0)
    E4 = (col4 == 4 * row4).astype(jnp.bfloat16)
    T4 = jnp.dot(T.astype(jnp.bfloat16), E4, preferred_element_type=f32)
    mn = jnp.minimum(R, G)
    s = mn + _roll(mn, -2)
    inter = s * _roll(s, -1)
    sg = G + _roll(G, -2)
    ga = sg * _roll(sg, -1)
    sr = R + _roll(R, -2)
    ra = sr * _roll(sr, -1)
    union = ga + ra - inter + 1e-7
    iou = (inter + 1.0) / (union + 1.0)
    rpn1_n = jnp.sum(jnp.where(T4 >= 0.5, 1.0 - iou, 0.0))
    rpn1_d = jnp.sum(T)

    Brr = br_ref[0]
    Bbb = bb_ref[0]
    Gv = gbt_ref[0]
    colb = jax.lax.broadcasted_iota(jnp.int32, (32, 128), 1)
    lm = colb % 4
    lo = lm < 2

    eG = _roll(Gv, -2) - Gv
    areaA = jnp.maximum(eG, 0.0) * jnp.maximum(_roll(eG, -1), 0.0)

    c = jnp.where(lo, jnp.maximum(Brr, Gv), jnp.minimum(Brr, Gv))
    wh = jnp.maximum(_roll(c, -2) - c, 0.0)
    inter_b = wh * _roll(wh, -1)
    eB = _roll(Brr, -2) - Brr
    areaB = jnp.maximum(eB, 0.0) * jnp.maximum(_roll(eB, -1), 0.0)
    union_b = areaA + areaB - inter_b + 1e-7
    iou4 = inter_b / jnp.maximum(union_b, 1e-12)

    cb = jnp.where(lo, jnp.maximum(Bbb, Gv), jnp.minimum(Bbb, Gv))
    whb = jnp.maximum(_roll(cb, -2) - cb, 0.0)
    inter_bb = whb * _roll(whb, -1)
    eBB = _roll(Bbb, -2) - Bbb
    areaBB = jnp.maximum(eBB, 0.0) * jnp.maximum(_roll(eBB, -1), 0.0)
    union_bb = areaA + areaBB - inter_bb + 1.0
    iou_bb4 = inter_bb / jnp.maximum(union_bb, 1e-12)

    pos4 = jnp.logical_and(iou4 >= _THR_POS, lm == 0)
    s_bb = jnp.sum(jnp.where(pos4, 1.0 - iou_bb4, 0.0))

    rowK = jax.lax.broadcasted_iota(jnp.int32, (128, 32), 0)
    colK = jax.lax.broadcasted_iota(jnp.int32, (128, 32), 1)
    K4 = (rowK == 4 * colK).astype(f32)
    iou_d = jnp.dot(iou4, K4, preferred_element_type=f32)
    pos_d = (iou_d >= _THR_POS).astype(f32)
    neg_d = (iou_d < _THR_NEG).astype(f32)
    pn = jnp.sum(pos_d)
    nn = jnp.sum(neg_d)

    xop = op_ref[0]
    bce = (jnp.maximum(xop, 0.0) - xop * iou_d
           + jnp.log1p(jnp.exp(-jnp.abs(xop))))
    s_op = jnp.sum(bce * pos_d)

    C = cf_ref[0]
    Cs = _roll(C, -1)
    lseE = jnp.maximum(C, Cs) + jnp.log1p(jnp.exp(-jnp.abs(C - Cs)))
    colc = jax.lax.broadcasted_iota(jnp.int32, (32, 256), 1)
    lseF = jnp.where(colc % 2 == 0, lseE, _roll(lseE, 1))
    nl = lseF - C
    rowc = jax.lax.broadcasted_iota(jnp.int32, (256, 32), 0)
    colc8 = jax.lax.broadcasted_iota(jnp.int32, (256, 32), 1)
    K0 = (rowc == 8 * colc8).astype(f32)
    K1 = (rowc == 8 * colc8 + 1).astype(f32)
    Kw = ((rowc == 8 * colc8 + 3) | (rowc == 8 * colc8 + 5)
          | (rowc == 8 * colc8 + 7)).astype(f32)
    U = jnp.dot(nl, K0, preferred_element_type=f32)
    V = jnp.dot(nl, K1, preferred_element_type=f32)
    Wn = jnp.dot(nl, Kw, preferred_element_type=f32)
    s_cfpos = jnp.sum(U * pos_d)
    s_cfnegb = jnp.sum(V * neg_d)
    s_cfneg = jnp.sum(Wn * pos_d)

    pnp = pn > 0.0
    loss_op = jnp.where(pnp, s_op / jnp.maximum(pn, 1.0), 0.0)
    loss_cf_pos = jnp.where(pnp, s_cfpos / jnp.maximum(pn, 1.0), 0.0)
    loss_cf_negb = jnp.where(nn > 0.0, s_cfnegb / jnp.maximum(nn, 1.0), 0.0)
    loss_cf_neg = jnp.where(pnp, s_cfneg / jnp.maximum(3.0 * pn, 1.0), 0.0)
    loss_bb = jnp.where(pnp, s_bb / jnp.maximum(pn, 1.0), 0.0)
    loss_i = jnp.where(
        pnp, loss_cf_pos + loss_cf_negb + loss_cf_neg + loss_bb + loss_op, 0.0)

    out_ref[0, 0] = out_ref[0, 0] + rpn0_s
    out_ref[0, 1] = out_ref[0, 1] + rpn1_n
    out_ref[0, 2] = out_ref[0, 2] + rpn1_d
    out_ref[0, 3] = out_ref[0, 3] + loss_i
    out_ref[0, 4] = out_ref[0, 4] + pn

    @pl.when(b == _B - 1)
    def _fin():
        a0 = out_ref[0, 0]
        a1 = out_ref[0, 1]
        a2 = out_ref[0, 2]
        a3 = out_ref[0, 3]
        rpn0 = a0 / float(_N_PIX)
        rpn1 = jnp.where(a2 > 0.0, a1 / jnp.maximum(a2, 1.0), 0.0)
        rcnn = a3 / float(_B)
        out_ref[0, 0] = rpn0 + rpn1 + rcnn
        out_ref[0, 1] = rpn0
        out_ref[0, 2] = rpn1
        out_ref[0, 3] = rcnn


def kernel(cl, re, cf, op, bb, br, gb, gr, gt):
    clr = cl.reshape(_B, _H, 2 * _W)
    rer = re.reshape(_B, _H, 4 * _W)
    grr = gr.reshape(_B, _H, 4 * _W)
    cfr = cf.reshape(_B, 32, 256)
    opr = op.reshape(_B, 32, 32)
    bbr = bb.reshape(_B, 32, 128)
    brr = br.reshape(_B, 32, 128)
    gbt = jnp.tile(gb, (1, 32)).reshape(_B, 1, 128)

    out = pl.pallas_call(
        _loss_kernel,
        grid=(_B,),
        in_specs=[
            pl.BlockSpec((1, _H, 2 * _W), lambda b: (b, 0, 0)),
            pl.BlockSpec((1, _H, 4 * _W), lambda b: (b, 0, 0)),
            pl.BlockSpec((1, _H, 4 * _W), lambda b: (b, 0, 0)),
            pl.BlockSpec((1, _H, _W), lambda b: (b, 0, 0)),
            pl.BlockSpec((1, 32, 256), lambda b: (b, 0, 0)),
            pl.BlockSpec((1, 32, 32), lambda b: (b, 0, 0)),
            pl.BlockSpec((1, 32, 128), lambda b: (b, 0, 0)),
            pl.BlockSpec((1, 32, 128), lambda b: (b, 0, 0)),
            pl.BlockSpec((1, 1, 128), lambda b: (b, 0, 0)),
        ],
        out_specs=pl.BlockSpec((1, 8), lambda b: (0, 0),
                               memory_space=pltpu.SMEM),
        out_shape=jax.ShapeDtypeStruct((1, 8), jnp.float32),
        compiler_params=pltpu.CompilerParams(
            dimension_semantics=("arbitrary",)),
    )(clr, rer, grr, gt, cfr, opr, bbr, brr, gbt)

    return (out[0, 0], out[0, 1], out[0, 2], out[0, 3], out[0, 4])
